# baseline (device time: 139571 ns/iter reference)
import os

import jax
import jax.numpy as jnp
from jax import lax
from jax.experimental import pallas as pl
from jax.experimental.pallas import tpu as pltpu

_KMODE = os.environ.get("KMODE", "full")

H = 16
S_PER = 1024
D = 128
SCALE = D ** -0.5
GROUP = 2
N_FLOWS = H // GROUP


def _compute(q_ref, kv_ref, kvo_ref, out_ref, h):
    q = (q_ref[0] * SCALE).astype(jnp.bfloat16)
    k1 = kv_ref[h][:, :D]
    v1 = kv_ref[h][:, D:]
    k2 = kvo_ref[h][:, :D]
    v2 = kvo_ref[h][:, D:]
    s1 = lax.dot_general(
        q, k1, (((1,), (1,)), ((), ())), preferred_element_type=jnp.float32
    )
    s2 = lax.dot_general(
        q, k2, (((1,), (1,)), ((), ())), preferred_element_type=jnp.float32
    )
    p1 = jnp.exp(s1).astype(jnp.bfloat16)
    p2 = jnp.exp(s2).astype(jnp.bfloat16)
    ones = jnp.ones((S_PER, 1), jnp.bfloat16)
    l = lax.dot_general(
        p1, ones, (((1,), (0,)), ((), ())), preferred_element_type=jnp.float32
    ) + lax.dot_general(
        p2, ones, (((1,), (0,)), ((), ())), preferred_element_type=jnp.float32
    )
    o = lax.dot_general(
        p1, v1, (((1,), (0,)), ((), ())), preferred_element_type=jnp.float32
    ) + lax.dot_general(
        p2, v2, (((1,), (0,)), ((), ())), preferred_element_type=jnp.float32
    )
    out_ref[0] = o / l


def _body(q_ref, kv_ref, out_ref, kvo_ref, dsend, drecv):
    h = pl.program_id(0)
    my_x = lax.axis_index("x")
    my_y = lax.axis_index("y")
    ynbr = (my_x, 1 - my_y)

    if _KMODE == "compute":
        _compute(q_ref, kv_ref, kv_ref, out_ref, h)
        return

    def flow(g):
        return pltpu.make_async_remote_copy(
            src_ref=kv_ref.at[pl.ds(GROUP * g, GROUP)],
            dst_ref=kvo_ref.at[pl.ds(GROUP * g, GROUP)],
            send_sem=dsend.at[g],
            recv_sem=drecv.at[g],
            device_id=ynbr,
            device_id_type=pl.DeviceIdType.MESH,
        )

    @pl.when(h == 0)
    def _start():
        barrier_sem = pltpu.get_barrier_semaphore()
        pl.semaphore_signal(
            barrier_sem, inc=1, device_id=ynbr,
            device_id_type=pl.DeviceIdType.MESH,
        )
        pl.semaphore_wait(barrier_sem, 1)
        for g in range(N_FLOWS):
            flow(g).start()

    @pl.when(h % GROUP == 0)
    def _wait_group():
        g = h // GROUP
        flow(g).wait_recv()
        flow(g).wait_send()

    if _KMODE == "comm":
        out_ref[0] = kvo_ref[h][:, :D].astype(jnp.float32)
    else:
        _compute(q_ref, kv_ref, kvo_ref, out_ref, h)


def kernel(Q, K, V):
    q = jnp.transpose(Q[0], (1, 0, 2))
    kv = jnp.concatenate(
        [jnp.transpose(K[0], (1, 0, 2)), jnp.transpose(V[0], (1, 0, 2))], axis=-1
    ).astype(jnp.bfloat16)

    out = pl.pallas_call(
        _body,
        grid=(H,),
        out_shape=jax.ShapeDtypeStruct((H, S_PER, D), jnp.float32),
        in_specs=[
            pl.BlockSpec((1, S_PER, D), lambda h: (h, 0, 0)),
            pl.BlockSpec(memory_space=pltpu.VMEM),
        ],
        out_specs=pl.BlockSpec((1, S_PER, D), lambda h: (h, 0, 0)),
        scratch_shapes=[
            pltpu.VMEM((H, S_PER, 2 * D), jnp.bfloat16),
            pltpu.SemaphoreType.DMA((N_FLOWS,)),
            pltpu.SemaphoreType.DMA((N_FLOWS,)),
        ],
        compiler_params=pltpu.CompilerParams(
            collective_id=None if _KMODE == "compute" else 0,
            vmem_limit_bytes=46 * 1024 * 1024,
        ),
    )(q, kv)

    return jnp.transpose(out, (1, 0, 2))[None]


# device time: 101423 ns/iter; 1.3761x vs baseline; 1.3761x over previous
import os

import jax
import jax.numpy as jnp
from jax import lax
from jax.experimental import pallas as pl
from jax.experimental.pallas import tpu as pltpu

_KMODE = os.environ.get("KMODE", "full")

H = 16
S_PER = 1024
D = 128
SCALE = D ** -0.5
H_MINE = H // 2


def _compute(q, kv, kvo):
    qb = (q * SCALE).astype(jnp.bfloat16)
    s1 = lax.dot_general(
        qb, kv[:, :D], (((1,), (1,)), ((), ())), preferred_element_type=jnp.float32
    )
    s2 = lax.dot_general(
        qb, kvo[:, :D], (((1,), (1,)), ((), ())), preferred_element_type=jnp.float32
    )
    p1 = jnp.exp(s1).astype(jnp.bfloat16)
    p2 = jnp.exp(s2).astype(jnp.bfloat16)
    ones = jnp.ones((S_PER, 1), jnp.bfloat16)
    l = lax.dot_general(
        p1, ones, (((1,), (0,)), ((), ())), preferred_element_type=jnp.float32
    ) + lax.dot_general(
        p2, ones, (((1,), (0,)), ((), ())), preferred_element_type=jnp.float32
    )
    o = lax.dot_general(
        p1, kv[:, D:], (((1,), (0,)), ((), ())), preferred_element_type=jnp.float32
    ) + lax.dot_general(
        p2, kvo[:, D:], (((1,), (0,)), ((), ())), preferred_element_type=jnp.float32
    )
    return o / l


def _body(q_ref, kv_ref, out_ref, kvo_ref, stage_ref, dsend, drecv, osend, orecv):
    s = pl.program_id(0)
    my_x = lax.axis_index("x")
    my_y = lax.axis_index("y")
    ynbr = (my_x, 1 - my_y)
    xnbr = (1 - my_x, my_y)
    base = my_x * H_MINE
    obase = (1 - my_x) * H_MINE

    if _KMODE == "compute":
        out_ref[base + s] = _compute(q_ref[base + s], kv_ref[base + s], kv_ref[base + s])
        return

    def kv_flow(g):
        return pltpu.make_async_remote_copy(
            src_ref=kv_ref.at[pl.ds(base + g, 1)],
            dst_ref=kvo_ref.at[pl.ds(g, 1)],
            send_sem=dsend.at[g],
            recv_sem=drecv.at[g],
            device_id=ynbr,
            device_id_type=pl.DeviceIdType.MESH,
        )

    def out_flow(g):
        return pltpu.make_async_remote_copy(
            src_ref=stage_ref.at[pl.ds(base + g, 1)],
            dst_ref=stage_ref.at[pl.ds(base + g, 1)],
            send_sem=osend.at[g],
            recv_sem=orecv.at[g],
            device_id=xnbr,
            device_id_type=pl.DeviceIdType.MESH,
        )

    @pl.when(s == 0)
    def _start():
        barrier_sem = pltpu.get_barrier_semaphore()
        for nbr in (ynbr, xnbr):
            pl.semaphore_signal(
                barrier_sem, inc=1, device_id=nbr,
                device_id_type=pl.DeviceIdType.MESH,
            )
        pl.semaphore_wait(barrier_sem, 2)
        for g in range(H_MINE):
            kv_flow(g).start()

    kv_flow(s).wait_recv()
    kv_flow(s).wait_send()

    if _KMODE == "comm":
        out_ref[base + s] = kvo_ref[s][:, :D].astype(jnp.float32)
    else:
        res = _compute(q_ref[base + s], kv_ref[base + s], kvo_ref[s])
        out_ref[base + s] = res
        stage_ref[base + s] = res.astype(jnp.bfloat16)

    out_flow(s).start()

    @pl.when(s == H_MINE - 1)
    def _finish():
        for g in range(H_MINE):
            out_flow(g).wait_send()
            pltpu.make_async_remote_copy(
                src_ref=stage_ref.at[pl.ds(obase + g, 1)],
                dst_ref=stage_ref.at[pl.ds(obase + g, 1)],
                send_sem=osend.at[g],
                recv_sem=orecv.at[g],
                device_id=xnbr,
                device_id_type=pl.DeviceIdType.MESH,
            ).wait_recv()
        out_ref[pl.ds(obase, H_MINE)] = stage_ref[pl.ds(obase, H_MINE)].astype(
            jnp.float32
        )


def kernel(Q, K, V):
    q = jnp.transpose(Q[0], (1, 0, 2))
    kv = jnp.concatenate(
        [jnp.transpose(K[0], (1, 0, 2)), jnp.transpose(V[0], (1, 0, 2))], axis=-1
    ).astype(jnp.bfloat16)

    out = pl.pallas_call(
        _body,
        grid=(H_MINE,),
        out_shape=jax.ShapeDtypeStruct((H, S_PER, D), jnp.float32),
        in_specs=[
            pl.BlockSpec(memory_space=pltpu.VMEM),
            pl.BlockSpec(memory_space=pltpu.VMEM),
        ],
        out_specs=pl.BlockSpec(memory_space=pltpu.VMEM),
        scratch_shapes=[
            pltpu.VMEM((H_MINE, S_PER, 2 * D), jnp.bfloat16),
            pltpu.VMEM((H, S_PER, D), jnp.bfloat16),
            pltpu.SemaphoreType.DMA((H_MINE,)),
            pltpu.SemaphoreType.DMA((H_MINE,)),
            pltpu.SemaphoreType.DMA((H_MINE,)),
            pltpu.SemaphoreType.DMA((H_MINE,)),
        ],
        compiler_params=pltpu.CompilerParams(
            collective_id=None if _KMODE == "compute" else 0,
            vmem_limit_bytes=50 * 1024 * 1024,
        ),
    )(q, kv)

    return jnp.transpose(out, (1, 0, 2))[None]


# device time: 100746 ns/iter; 1.3854x vs baseline; 1.0067x over previous
import os

import jax
import jax.numpy as jnp
from jax import lax
from jax.experimental import pallas as pl
from jax.experimental.pallas import tpu as pltpu

_KMODE = os.environ.get("KMODE", "full")

H = 16
S_PER = 1024
D = 128
SCALE = D ** -0.5
H_MINE = H // 2
GROUP = 2
N_FLOWS = H_MINE // GROUP


def _compute(q, kv, kvo):
    qb = (q * SCALE).astype(jnp.bfloat16)
    s1 = lax.dot_general(
        qb, kv[:, :D], (((1,), (1,)), ((), ())), preferred_element_type=jnp.float32
    )
    s2 = lax.dot_general(
        qb, kvo[:, :D], (((1,), (1,)), ((), ())), preferred_element_type=jnp.float32
    )
    p1 = jnp.exp(s1).astype(jnp.bfloat16)
    p2 = jnp.exp(s2).astype(jnp.bfloat16)
    ones = jnp.ones((S_PER, 1), jnp.bfloat16)
    l = lax.dot_general(
        p1, ones, (((1,), (0,)), ((), ())), preferred_element_type=jnp.float32
    ) + lax.dot_general(
        p2, ones, (((1,), (0,)), ((), ())), preferred_element_type=jnp.float32
    )
    o = lax.dot_general(
        p1, kv[:, D:], (((1,), (0,)), ((), ())), preferred_element_type=jnp.float32
    ) + lax.dot_general(
        p2, kvo[:, D:], (((1,), (0,)), ((), ())), preferred_element_type=jnp.float32
    )
    return (o / l).astype(jnp.bfloat16)


def _body(q_ref, kv_ref, out_ref, kvo_ref, dsend, drecv, osend, orecv):
    s = pl.program_id(0)
    my_x = lax.axis_index("x")
    my_y = lax.axis_index("y")
    ynbr = (my_x, 1 - my_y)
    xnbr = (1 - my_x, my_y)
    base = my_x * H_MINE

    if _KMODE == "compute":
        out_ref[base + s] = _compute(
            q_ref[base + s], kv_ref[base + s], kv_ref[base + s]
        )
        return

    def kv_flow(g):
        return pltpu.make_async_remote_copy(
            src_ref=kv_ref.at[pl.ds(base + GROUP * g, GROUP)],
            dst_ref=kvo_ref.at[pl.ds(GROUP * g, GROUP)],
            send_sem=dsend.at[g],
            recv_sem=drecv.at[g],
            device_id=ynbr,
            device_id_type=pl.DeviceIdType.MESH,
        )

    def out_flow(g):
        slc = out_ref.at[pl.ds(base + GROUP * g, GROUP)]
        return pltpu.make_async_remote_copy(
            src_ref=slc,
            dst_ref=slc,
            send_sem=osend.at[g],
            recv_sem=orecv.at[g],
            device_id=xnbr,
            device_id_type=pl.DeviceIdType.MESH,
        )

    @pl.when(s == 0)
    def _start():
        barrier_sem = pltpu.get_barrier_semaphore()
        for nbr in (ynbr, xnbr):
            pl.semaphore_signal(
                barrier_sem, inc=1, device_id=nbr,
                device_id_type=pl.DeviceIdType.MESH,
            )
        pl.semaphore_wait(barrier_sem, 2)
        for g in range(N_FLOWS):
            kv_flow(g).start()

    @pl.when(s % GROUP == 0)
    def _wait_group():
        g = s // GROUP
        kv_flow(g).wait_recv()
        kv_flow(g).wait_send()

    if _KMODE == "comm":
        out_ref[base + s] = kvo_ref[s][:, :D]
    else:
        out_ref[base + s] = _compute(q_ref[base + s], kv_ref[base + s], kvo_ref[s])

    @pl.when(s % GROUP == GROUP - 1)
    def _ship_group():
        out_flow(s // GROUP).start()

    @pl.when(s == H_MINE - 1)
    def _finish():
        for g in range(N_FLOWS):
            out_flow(g).wait_send()
            out_flow(g).wait_recv()


def kernel(Q, K, V):
    q = jnp.transpose(Q[0], (1, 0, 2))
    kv = jnp.transpose(
        jnp.concatenate(
            [K[0].astype(jnp.bfloat16), V[0].astype(jnp.bfloat16)], axis=-1
        ),
        (1, 0, 2),
    )

    out = pl.pallas_call(
        _body,
        grid=(H_MINE,),
        out_shape=jax.ShapeDtypeStruct((H, S_PER, D), jnp.bfloat16),
        in_specs=[
            pl.BlockSpec(memory_space=pltpu.VMEM),
            pl.BlockSpec(memory_space=pltpu.VMEM),
        ],
        out_specs=pl.BlockSpec(memory_space=pltpu.VMEM),
        scratch_shapes=[
            pltpu.VMEM((H_MINE, S_PER, 2 * D), jnp.bfloat16),
            pltpu.SemaphoreType.DMA((N_FLOWS,)),
            pltpu.SemaphoreType.DMA((N_FLOWS,)),
            pltpu.SemaphoreType.DMA((N_FLOWS,)),
            pltpu.SemaphoreType.DMA((N_FLOWS,)),
        ],
        compiler_params=pltpu.CompilerParams(
            collective_id=None if _KMODE == "compute" else 0,
            vmem_limit_bytes=50 * 1024 * 1024,
        ),
    )(q, kv)

    return jnp.transpose(out, (1, 0, 2))[None]


# device time: 87754 ns/iter; 1.5905x vs baseline; 1.1481x over previous
import os

import jax
import jax.numpy as jnp
from jax import lax
from jax.experimental import pallas as pl
from jax.experimental.pallas import tpu as pltpu

_KMODE = os.environ.get("KMODE", "full")

H = 16
S_PER = 1024
D = 128
SCALE = D ** -0.5
H_MINE = H // 2
GROUP = 2
N_FLOWS = H_MINE // GROUP


def _compute(q, kv, kvo):
    qb = (q * SCALE).astype(jnp.bfloat16)
    s1 = lax.dot_general(
        qb, kv[:, :D], (((1,), (1,)), ((), ())), preferred_element_type=jnp.float32
    )
    s2 = lax.dot_general(
        qb, kvo[:, :D], (((1,), (1,)), ((), ())), preferred_element_type=jnp.float32
    )
    p1 = jnp.exp(s1).astype(jnp.bfloat16)
    p2 = jnp.exp(s2).astype(jnp.bfloat16)
    ones = jnp.ones((S_PER, 1), jnp.bfloat16)
    l = lax.dot_general(
        p1, ones, (((1,), (0,)), ((), ())), preferred_element_type=jnp.float32
    ) + lax.dot_general(
        p2, ones, (((1,), (0,)), ((), ())), preferred_element_type=jnp.float32
    )
    o = lax.dot_general(
        p1, kv[:, D:], (((1,), (0,)), ((), ())), preferred_element_type=jnp.float32
    ) + lax.dot_general(
        p2, kvo[:, D:], (((1,), (0,)), ((), ())), preferred_element_type=jnp.float32
    )
    return (o / l).astype(jnp.bfloat16)


def _body(q_ref, kv_ref, out_ref, kvo_ref, dsend, drecv, osend, orecv):
    s = pl.program_id(0)
    my_x = lax.axis_index("x")
    my_y = lax.axis_index("y")
    ynbr = (my_x, 1 - my_y)
    xnbr = (1 - my_x, my_y)
    base = my_x * H_MINE

    if _KMODE == "compute":
        out_ref[base + s] = _compute(q_ref[0], kv_ref[s], kv_ref[s])
        return

    def kv_flow(g):
        return pltpu.make_async_remote_copy(
            src_ref=kv_ref.at[pl.ds(GROUP * g, GROUP)],
            dst_ref=kvo_ref.at[pl.ds(GROUP * g, GROUP)],
            send_sem=dsend.at[g],
            recv_sem=drecv.at[g],
            device_id=ynbr,
            device_id_type=pl.DeviceIdType.MESH,
        )

    def out_flow(g):
        slc = out_ref.at[pl.ds(base + GROUP * g, GROUP)]
        return pltpu.make_async_remote_copy(
            src_ref=slc,
            dst_ref=slc,
            send_sem=osend.at[g],
            recv_sem=orecv.at[g],
            device_id=xnbr,
            device_id_type=pl.DeviceIdType.MESH,
        )

    @pl.when(s == 0)
    def _start():
        barrier_sem = pltpu.get_barrier_semaphore()
        for nbr in (ynbr, xnbr):
            pl.semaphore_signal(
                barrier_sem, inc=1, device_id=nbr,
                device_id_type=pl.DeviceIdType.MESH,
            )
        pl.semaphore_wait(barrier_sem, 2)
        for g in range(N_FLOWS):
            kv_flow(g).start()

    @pl.when(s % GROUP == 0)
    def _wait_group():
        g = s // GROUP
        kv_flow(g).wait_recv()
        kv_flow(g).wait_send()

    if _KMODE == "comm":
        out_ref[base + s] = kvo_ref[s][:, :D]
    else:
        out_ref[base + s] = _compute(q_ref[0], kv_ref[s], kvo_ref[s])

    @pl.when(s % GROUP == GROUP - 1)
    def _ship_group():
        out_flow(s // GROUP).start()

    @pl.when(s == H_MINE - 1)
    def _finish():
        for g in range(N_FLOWS):
            out_flow(g).wait_send()
            out_flow(g).wait_recv()


def kernel(Q, K, V):
    my_x = lax.axis_index("x")
    base_h = my_x * H_MINE
    qm = jnp.transpose(
        lax.dynamic_slice(Q[0], (0, base_h, 0), (S_PER, H_MINE, D)), (1, 0, 2)
    )
    kv = jnp.transpose(
        jnp.concatenate(
            [
                lax.dynamic_slice(K[0], (0, base_h, 0), (S_PER, H_MINE, D)),
                lax.dynamic_slice(V[0], (0, base_h, 0), (S_PER, H_MINE, D)),
            ],
            axis=-1,
        ).astype(jnp.bfloat16),
        (1, 0, 2),
    )

    out = pl.pallas_call(
        _body,
        grid=(H_MINE,),
        out_shape=jax.ShapeDtypeStruct((H, S_PER, D), jnp.bfloat16),
        in_specs=[
            pl.BlockSpec((1, S_PER, D), lambda s: (s, 0, 0)),
            pl.BlockSpec(memory_space=pltpu.VMEM),
        ],
        out_specs=pl.BlockSpec(memory_space=pltpu.VMEM),
        scratch_shapes=[
            pltpu.VMEM((H_MINE, S_PER, 2 * D), jnp.bfloat16),
            pltpu.SemaphoreType.DMA((N_FLOWS,)),
            pltpu.SemaphoreType.DMA((N_FLOWS,)),
            pltpu.SemaphoreType.DMA((N_FLOWS,)),
            pltpu.SemaphoreType.DMA((N_FLOWS,)),
        ],
        compiler_params=pltpu.CompilerParams(
            collective_id=None if _KMODE == "compute" else 0,
            vmem_limit_bytes=50 * 1024 * 1024,
        ),
    )(qm, kv)

    return jnp.transpose(out, (1, 0, 2))[None]
